# Initial kernel scaffold; baseline (speedup 1.0000x reference)
import jax
import jax.numpy as jnp
from jax.experimental import pallas as pl


def kernel(X0, X1, X2, L0_indices, L0_values, L1_indices, L1_values, L2_indices, L2_values, batch0, batch1, batch2, W0_1, b0_1, W0_2, b0_2, W0_3, b0_3, W1_1, b1_1, W1_2, b1_2, W1_3, b1_3, W2_1, b2_1, W2_2, b2_2, W2_3, b2_3, Wl0, bl0, Wl1, bl1, Wl2, bl2, Wc, bc):
    raise NotImplementedError("write your pallas kernel here")


# speedup vs baseline: 3.7728x; 3.7728x over previous
import functools

import jax
import jax.numpy as jnp
from jax import lax
from jax.experimental import pallas as pl
from jax.experimental.pallas import tpu as pltpu
from jax.experimental.pallas import tpu_sc as plsc

N0, N1, N2 = 100000, 400000, 200000
G = 128
E = 640
EG = E // 16
ZB = 512


def _chunk_starts(total, size):
    starts = []
    i = 0
    while i + size <= total:
        starts.append(i)
        i += size
    if i < total:
        starts.append(total - size)
    return starts


@functools.lru_cache(maxsize=None)
def _make_spmm(n, nnz, d, r, p):
    assert n == 2 * p * r and nnz % E == 0
    b_tot = nnz // E
    nb = -(-b_tot // 16)
    stripe = r // 16
    nh = d // 16
    mesh = plsc.VectorSubcoreMesh(core_axis_name="c", subcore_axis_name="s")

    @functools.partial(
        pl.kernel,
        out_type=jax.ShapeDtypeStruct((n, d), jnp.float32),
        mesh=mesh,
        scratch_types=[
            pltpu.VMEM((ZB, d), jnp.float32),
            pltpu.VMEM((E,), jnp.int32),
            pltpu.VMEM((E,), jnp.int32),
            pltpu.VMEM((E,), jnp.float32),
            pltpu.VMEM((E, d), jnp.float32),
        ] + [pltpu.VMEM((128,), jnp.int32) for _ in range(E // 128)] + [
            pltpu.VMEM_SHARED((r, d), jnp.float32),
        ],
        compiler_params=pltpu.CompilerParams(use_tc_tiling_on_sc=False),
    )
    def spmm(x_hbm, rows_hbm, cols_hbm, vals_hbm, out_hbm,
             zbuf, rbuf, cbuf, vbuf, rowsb, *rest):
        lidxs, acc = rest[:-1], rest[-1]
        cid = lax.axis_index("c")
        sid = lax.axis_index("s")

        def zero_zbuf(i, _):
            for h in range(nh):
                zbuf[i, pl.ds(16 * h, 16)] = jnp.zeros((16,), jnp.float32)
            return _
        lax.fori_loop(0, ZB, zero_zbuf, None)

        nchunk = r // ZB
        rounds = -(-nchunk // 16)
        tail = r % ZB != 0

        def sweep(lo, flush):
            def rnd(j, _):
                ch = j * 16 + sid

                @pl.when(ch < nchunk)
                def _():
                    st = pl.multiple_of(ch * ZB, ZB)
                    if flush:
                        pltpu.sync_copy(
                            acc.at[pl.ds(st, ZB)],
                            out_hbm.at[pl.ds(pl.multiple_of(lo + st, 8), ZB)])
                    else:
                        pltpu.sync_copy(zbuf, acc.at[pl.ds(st, ZB)])
                return _
            lax.fori_loop(0, rounds, rnd, None)
            if tail:
                @pl.when(sid == 0)
                def _():
                    st = r - ZB
                    if flush:
                        pltpu.sync_copy(
                            acc.at[pl.ds(st, ZB)],
                            out_hbm.at[pl.ds(pl.multiple_of(lo + st, 8), ZB)])
                    else:
                        pltpu.sync_copy(zbuf, acc.at[pl.ds(st, ZB)])

        for pp in range(p):
            win = 2 * pp + cid
            lo = win * r
            sweep(lo, flush=False)
            plsc.subcore_barrier()

            def block(j, _):
                ib = j * 16 + sid

                @pl.when(ib < b_tot)
                def _():
                    base = ib * E
                    pltpu.sync_copy(rows_hbm.at[pl.ds(base, E)], rbuf)
                    pltpu.sync_copy(cols_hbm.at[pl.ds(base, E)], cbuf)
                    pltpu.sync_copy(vals_hbm.at[pl.ds(base, E)], vbuf)
                    pltpu.sync_copy(x_hbm.at[cbuf], rowsb)

                    for k in range(E // 128):
                        def group(g2, _, k=k):
                            g = k * 8 + g2
                            r16 = rbuf[pl.ds(g * 16, 16)]
                            v16 = vbuf[pl.ds(g * 16, 16)]
                            inw = (r16 >= lo) & (r16 < lo + r)
                            v16 = jnp.where(inw, v16, 0.0)
                            loc = jnp.where(inw, r16 - lo, r16 & 8191)
                            lidxs[k][pl.ds(g2 * 16, 16)] = loc
                            for e in range(16):
                                vb = jnp.broadcast_to(v16[e], (16,))
                                ge = g * 16 + e
                                for h in range(nh):
                                    sl = pl.ds(16 * h, 16)
                                    rowsb[ge, sl] = rowsb[ge, sl] * vb
                            return _
                        lax.fori_loop(0, 8, group, None)
                        pltpu.sync_copy(rowsb.at[pl.ds(k * 128, 128)],
                                        acc.at[lidxs[k]], add=True)
                return _
            lax.fori_loop(0, nb, block, None)
            plsc.subcore_barrier()
            sweep(lo, flush=True)
            plsc.subcore_barrier()

    return spmm


_BT = 2000


def _dense_lr(t, w, b):
    n, din = t.shape
    dout = w.shape[1]

    def body(t_ref, w_ref, b_ref, o_ref):
        y = jnp.dot(t_ref[...], w_ref[...],
                    preferred_element_type=jnp.float32) + b_ref[...]
        o_ref[...] = jnp.where(y >= 0, y, 0.01 * y)

    return pl.pallas_call(
        body,
        grid=(n // _BT,),
        in_specs=[
            pl.BlockSpec((_BT, din), lambda i: (i, 0)),
            pl.BlockSpec((din, dout), lambda i: (0, 0)),
            pl.BlockSpec((1, dout), lambda i: (0, 0)),
        ],
        out_specs=pl.BlockSpec((_BT, dout), lambda i: (i, 0)),
        out_shape=jax.ShapeDtypeStruct((n, dout), jnp.float32),
    )(t, w, b[None, :])


def _pool_proj(h1, h2, h3, wl, batch2d):
    n = h1.shape[0]

    def body(h1_ref, h2_ref, h3_ref, b_ref, wl_ref, o_ref):
        i = pl.program_id(0)
        y = (jnp.dot(h1_ref[...], wl_ref[0:32, :],
                     preferred_element_type=jnp.float32)
             + jnp.dot(h2_ref[...], wl_ref[32:64, :],
                       preferred_element_type=jnp.float32)
             + jnp.dot(h3_ref[...], wl_ref[64:96, :],
                       preferred_element_type=jnp.float32))
        yext = jnp.concatenate(
            [y, jnp.ones((_BT, 1), jnp.float32),
             jnp.zeros((_BT, 5), jnp.float32)], axis=1)
        onehot = (b_ref[...] == lax.broadcasted_iota(jnp.int32, (1, G), 1))
        contrib = lax.dot_general(onehot.astype(jnp.float32), yext,
                                  (((0,), (0,)), ((), ())),
                                  preferred_element_type=jnp.float32)

        @pl.when(i == 0)
        def _():
            o_ref[...] = jnp.zeros_like(o_ref)
        o_ref[...] += contrib

    return pl.pallas_call(
        body,
        grid=(n // _BT,),
        in_specs=[
            pl.BlockSpec((_BT, 32), lambda i: (i, 0)),
            pl.BlockSpec((_BT, 32), lambda i: (i, 0)),
            pl.BlockSpec((_BT, 32), lambda i: (i, 0)),
            pl.BlockSpec((_BT, 1), lambda i: (i, 0)),
            pl.BlockSpec((96, 10), lambda i: (0, 0)),
        ],
        out_specs=pl.BlockSpec((G, 16), lambda i: (0, 0)),
        out_shape=jax.ShapeDtypeStruct((G, 16), jnp.float32),
    )(h1, h2, h3, batch2d, wl)


def _final(p0, p1, p2, bl0, bl1, bl2, wc, bc):
    def body(p0_ref, p1_ref, p2_ref, bl_ref, wc_ref, bc_ref, o_ref):
        ms = []
        for k, p_ref in enumerate((p0_ref, p1_ref, p2_ref)):
            s = p_ref[:, 0:10]
            cnt = p_ref[:, 10:11]
            ms.append((s + cnt * bl_ref[k:k + 1, :]) / jnp.maximum(cnt, 1.0))
        feats = jnp.concatenate(ms, axis=1)
        logits = jnp.dot(feats, wc_ref[...],
                         preferred_element_type=jnp.float32) + bc_ref[...]
        z = logits - jnp.max(logits, axis=1, keepdims=True)
        ez = jnp.exp(z)
        o_ref[...] = ez / jnp.sum(ez, axis=1, keepdims=True)

    bl = jnp.stack([bl0, bl1, bl2], axis=0)
    return pl.pallas_call(
        body,
        out_shape=jax.ShapeDtypeStruct((G, 10), jnp.float32),
    )(p0, p1, p2, bl, wc, bc[None, :])


_CFG = {
    0: (N0, 1600000, 50000, 1, 25000, 2),
    1: (N1, 2000000, 50000, 4, 40000, 5),
    2: (N2, 800000, 50000, 2, 25000, 4),
}


def _order(x, l_idx, l_val, batch, w1, b1, w2, b2, w3, b3, wl, cfg):
    n, nnz, r16, p16, r32, p32 = cfg
    rows = l_idx[0].astype(jnp.int32)
    cols = l_idx[1].astype(jnp.int32)
    spmm16 = _make_spmm(n, nnz, 16, r16, p16)
    spmm32 = _make_spmm(n, nnz, 32, r32, p32)
    t1 = spmm16(x, rows, cols, l_val)
    h1 = _dense_lr(t1, w1, b1)
    t2 = spmm32(h1, rows, cols, l_val)
    h2 = _dense_lr(t2, w2, b2)
    t3 = spmm32(h2, rows, cols, l_val)
    h3 = _dense_lr(t3, w3, b3)
    return _pool_proj(h1, h2, h3, wl, batch[:, None].astype(jnp.int32))


def kernel(X0, X1, X2, L0_indices, L0_values, L1_indices, L1_values,
           L2_indices, L2_values, batch0, batch1, batch2,
           W0_1, b0_1, W0_2, b0_2, W0_3, b0_3,
           W1_1, b1_1, W1_2, b1_2, W1_3, b1_3,
           W2_1, b2_1, W2_2, b2_2, W2_3, b2_3,
           Wl0, bl0, Wl1, bl1, Wl2, bl2, Wc, bc):
    p0 = _order(X0, L0_indices, L0_values, batch0,
                W0_1, b0_1, W0_2, b0_2, W0_3, b0_3, Wl0, _CFG[0])
    p1 = _order(X1, L1_indices, L1_values, batch1,
                W1_1, b1_1, W1_2, b1_2, W1_3, b1_3, Wl1, _CFG[1])
    p2 = _order(X2, L2_indices, L2_values, batch2,
                W2_1, b2_1, W2_2, b2_2, W2_3, b2_3, Wl2, _CFG[2])
    return _final(p0, p1, p2, bl0, bl1, bl2, Wc, bc)


# speedup vs baseline: 9.8174x; 2.6022x over previous
import functools

import jax
import jax.numpy as jnp
from jax import lax
from jax.experimental import pallas as pl
from jax.experimental.pallas import tpu as pltpu
from jax.experimental.pallas import tpu_sc as plsc

N0, N1, N2 = 100000, 400000, 200000
G = 128
E = 640
EG = E // 16
ZB = 128


def _chunk_starts(total, size):
    starts = []
    i = 0
    while i + size <= total:
        starts.append(i)
        i += size
    if i < total:
        starts.append(total - size)
    return starts


@functools.lru_cache(maxsize=None)
def _make_spmm(n, nnz, d, r, p):
    assert n == 2 * p * r and nnz % E == 0
    b_tot = nnz // E
    nb = -(-b_tot // 16)
    stripe = r // 16
    nh = d // 16
    mesh = plsc.VectorSubcoreMesh(core_axis_name="c", subcore_axis_name="s")

    nk = E // 128

    @functools.partial(
        pl.kernel,
        out_type=jax.ShapeDtypeStruct((n, d), jnp.float32),
        mesh=mesh,
        scratch_types=[
            pltpu.VMEM((ZB, d), jnp.float32),
            pltpu.VMEM((3, E), jnp.int32),
            pltpu.VMEM((3, E), jnp.int32),
            pltpu.VMEM((E, d), jnp.float32),
            pltpu.VMEM((E, d), jnp.float32),
        ] + [pltpu.VMEM((128,), jnp.int32) for _ in range(2 * nk)] + [
            pltpu.VMEM_SHARED((r, d), jnp.float32),
            pltpu.SemaphoreType.DMA, pltpu.SemaphoreType.DMA,
            pltpu.SemaphoreType.DMA, pltpu.SemaphoreType.DMA,
            pltpu.SemaphoreType.DMA, pltpu.SemaphoreType.DMA,
        ],
        compiler_params=pltpu.CompilerParams(use_tc_tiling_on_sc=False,
                                             needs_layout_passes=False),
    )
    def spmm(x_hbm, e3_hbm, out_hbm, zbuf, *rest):
        ebuf = rest[0:2]
        rowsb = rest[2:4]
        lidxs = (rest[4:4 + nk], rest[4 + nk:4 + 2 * nk])
        acc = rest[4 + 2 * nk]
        esem = rest[5 + 2 * nk:7 + 2 * nk]
        gsem = rest[7 + 2 * nk:9 + 2 * nk]
        ssem = rest[9 + 2 * nk:11 + 2 * nk]
        cid = lax.axis_index("c")
        sid = lax.axis_index("s")

        def zero_zbuf(i, _):
            for h in range(nh):
                zbuf[i, pl.ds(16 * h, 16)] = jnp.zeros((16,), jnp.float32)
            return _
        lax.fori_loop(0, ZB, zero_zbuf, None)

        nchunk = r // ZB
        rounds = -(-nchunk // 16)
        tail = r % ZB != 0

        def sweep(lo, flush):
            def rnd(j, _):
                ch = j * 16 + sid

                @pl.when(ch < nchunk)
                def _():
                    st = pl.multiple_of(ch * ZB, ZB)
                    if flush:
                        pltpu.sync_copy(
                            acc.at[pl.ds(st, ZB)],
                            out_hbm.at[pl.ds(pl.multiple_of(lo + st, 8), ZB)])
                    else:
                        pltpu.sync_copy(zbuf, acc.at[pl.ds(st, ZB)])
                return _
            lax.fori_loop(0, rounds, rnd, None)
            if tail:
                @pl.when(sid == 0)
                def _():
                    st = r - ZB
                    if flush:
                        pltpu.sync_copy(
                            acc.at[pl.ds(st, ZB)],
                            out_hbm.at[pl.ds(pl.multiple_of(lo + st, 8), ZB)])
                    else:
                        pltpu.sync_copy(zbuf, acc.at[pl.ds(st, ZB)])

        def valid(j):
            return (j >= 0) & (j * 16 + sid < b_tot)

        def start_edges(j, q):
            @pl.when(valid(j))
            def _():
                pltpu.async_copy(e3_hbm.at[j * 16 + sid], ebuf[q], esem[q])

        def wait_edges(j, q):
            @pl.when(valid(j))
            def _():
                pltpu.make_async_copy(e3_hbm.at[0], ebuf[q], esem[q]).wait()

        def start_gather(j, q):
            @pl.when(valid(j))
            def _():
                pltpu.async_copy(x_hbm.at[ebuf[q].at[1]], rowsb[q], gsem[q])

        def wait_gather(j, q):
            @pl.when(valid(j))
            def _():
                pltpu.make_async_copy(x_hbm.at[ebuf[q].at[1]], rowsb[q],
                                      gsem[q]).wait()

        def wait_scatter(j, q, acc_at):
            @pl.when(valid(j))
            def _():
                for k in range(nk):
                    pltpu.make_async_copy(rowsb[q].at[pl.ds(k * 128, 128)],
                                          acc_at(q, k), ssem[q]).wait()

        def make_compute(lo):
            def compute(j, q):
                @pl.when(valid(j))
                def _():
                    for k in range(nk):
                        def group(g2, _, k=k):
                            g = k * 8 + g2
                            r16 = ebuf[q][0, pl.ds(g * 16, 16)]
                            v16 = plsc.bitcast(ebuf[q][2, pl.ds(g * 16, 16)],
                                               jnp.float32)
                            inw = (r16 >= lo) & (r16 < lo + r)
                            v16 = jnp.where(inw, v16, 0.0)
                            loc = jnp.where(inw, r16 - lo, r16 & 8191)
                            lidxs[q][k][pl.ds(g2 * 16, 16)] = loc
                            for e in range(16):
                                vb = jnp.broadcast_to(v16[e], (16,))
                                ge = g * 16 + e
                                for h in range(nh):
                                    sl = pl.ds(16 * h, 16)
                                    rowsb[q][ge, sl] = rowsb[q][ge, sl] * vb
                            return _
                        lax.fori_loop(0, 8, group, None)
                        pltpu.async_copy(rowsb[q].at[pl.ds(k * 128, 128)],
                                         acc.at[lidxs[q][k]], ssem[q],
                                         add=True)
            return compute

        np_ = nb // 2 + 1

        for pp in range(p):
            win = 2 * pp + cid
            lo = win * r
            sweep(lo, flush=False)
            plsc.subcore_barrier()
            compute = make_compute(lo)

            def acc_at(q, k):
                return acc.at[lidxs[q][k]]

            def half(j, pq, oq):
                wait_edges(j, pq)
                wait_scatter(j - 2, pq, acc_at)
                start_gather(j, pq)
                wait_gather(j - 1, oq)
                compute(j - 1, oq)
                start_edges(j + 1, oq)

            start_edges(0, 0)

            def piter(i, _):
                half(2 * i, 0, 1)
                half(2 * i + 1, 1, 0)
                return _
            lax.fori_loop(0, np_, piter, None)
            jmax = 2 * np_ - 1
            wait_scatter(jmax - 1, 0, acc_at)
            wait_scatter(jmax, 1, acc_at)
            plsc.subcore_barrier()
            sweep(lo, flush=True)
            plsc.subcore_barrier()

    return spmm


_BT = 2000


def _dense_lr(t, w, b):
    n, din = t.shape
    dout = w.shape[1]

    def body(t_ref, w_ref, b_ref, o_ref):
        y = jnp.dot(t_ref[...], w_ref[...],
                    preferred_element_type=jnp.float32) + b_ref[...]
        o_ref[...] = jnp.where(y >= 0, y, 0.01 * y)

    return pl.pallas_call(
        body,
        grid=(n // _BT,),
        in_specs=[
            pl.BlockSpec((_BT, din), lambda i: (i, 0)),
            pl.BlockSpec((din, dout), lambda i: (0, 0)),
            pl.BlockSpec((1, dout), lambda i: (0, 0)),
        ],
        out_specs=pl.BlockSpec((_BT, dout), lambda i: (i, 0)),
        out_shape=jax.ShapeDtypeStruct((n, dout), jnp.float32),
    )(t, w, b[None, :])


def _pool_proj(h1, h2, h3, wl, batch2d):
    n = h1.shape[0]

    def body(h1_ref, h2_ref, h3_ref, b_ref, wl_ref, o_ref):
        i = pl.program_id(0)
        y = (jnp.dot(h1_ref[...], wl_ref[0:32, :],
                     preferred_element_type=jnp.float32)
             + jnp.dot(h2_ref[...], wl_ref[32:64, :],
                       preferred_element_type=jnp.float32)
             + jnp.dot(h3_ref[...], wl_ref[64:96, :],
                       preferred_element_type=jnp.float32))
        yext = jnp.concatenate(
            [y, jnp.ones((_BT, 1), jnp.float32),
             jnp.zeros((_BT, 5), jnp.float32)], axis=1)
        onehot = (b_ref[...] == lax.broadcasted_iota(jnp.int32, (1, G), 1))
        contrib = lax.dot_general(onehot.astype(jnp.float32), yext,
                                  (((0,), (0,)), ((), ())),
                                  preferred_element_type=jnp.float32)

        @pl.when(i == 0)
        def _():
            o_ref[...] = jnp.zeros_like(o_ref)
        o_ref[...] += contrib

    return pl.pallas_call(
        body,
        grid=(n // _BT,),
        in_specs=[
            pl.BlockSpec((_BT, 32), lambda i: (i, 0)),
            pl.BlockSpec((_BT, 32), lambda i: (i, 0)),
            pl.BlockSpec((_BT, 32), lambda i: (i, 0)),
            pl.BlockSpec((_BT, 1), lambda i: (i, 0)),
            pl.BlockSpec((96, 10), lambda i: (0, 0)),
        ],
        out_specs=pl.BlockSpec((G, 16), lambda i: (0, 0)),
        out_shape=jax.ShapeDtypeStruct((G, 16), jnp.float32),
    )(h1, h2, h3, batch2d, wl)


def _final(p0, p1, p2, bl0, bl1, bl2, wc, bc):
    def body(p0_ref, p1_ref, p2_ref, bl_ref, wc_ref, bc_ref, o_ref):
        ms = []
        for k, p_ref in enumerate((p0_ref, p1_ref, p2_ref)):
            s = p_ref[:, 0:10]
            cnt = p_ref[:, 10:11]
            ms.append((s + cnt * bl_ref[k:k + 1, :]) / jnp.maximum(cnt, 1.0))
        feats = jnp.concatenate(ms, axis=1)
        logits = jnp.dot(feats, wc_ref[...],
                         preferred_element_type=jnp.float32) + bc_ref[...]
        z = logits - jnp.max(logits, axis=1, keepdims=True)
        ez = jnp.exp(z)
        o_ref[...] = ez / jnp.sum(ez, axis=1, keepdims=True)

    bl = jnp.stack([bl0, bl1, bl2], axis=0)
    return pl.pallas_call(
        body,
        out_shape=jax.ShapeDtypeStruct((G, 10), jnp.float32),
    )(p0, p1, p2, bl, wc, bc[None, :])


_CFG = {
    0: (N0, 1600000, 50000, 1, 25000, 2),
    1: (N1, 2000000, 100000, 2, 40000, 5),
    2: (N2, 800000, 100000, 1, 25000, 4),
}


def _order(x, l_idx, l_val, batch, w1, b1, w2, b2, w3, b3, wl, cfg):
    n, nnz, r16, p16, r32, p32 = cfg
    nbt = nnz // E
    rows = l_idx[0].astype(jnp.int32).reshape(nbt, E)
    cols = l_idx[1].astype(jnp.int32).reshape(nbt, E)
    vals = lax.bitcast_convert_type(l_val, jnp.int32).reshape(nbt, E)
    e3 = jnp.stack([rows, cols, vals], axis=1)
    spmm16 = _make_spmm(n, nnz, 16, r16, p16)
    spmm32 = _make_spmm(n, nnz, 32, r32, p32)
    t1 = spmm16(x, e3)
    h1 = _dense_lr(t1, w1, b1)
    t2 = spmm32(h1, e3)
    h2 = _dense_lr(t2, w2, b2)
    t3 = spmm32(h2, e3)
    h3 = _dense_lr(t3, w3, b3)
    return _pool_proj(h1, h2, h3, wl, batch[:, None].astype(jnp.int32))


def kernel(X0, X1, X2, L0_indices, L0_values, L1_indices, L1_values,
           L2_indices, L2_values, batch0, batch1, batch2,
           W0_1, b0_1, W0_2, b0_2, W0_3, b0_3,
           W1_1, b1_1, W1_2, b1_2, W1_3, b1_3,
           W2_1, b2_1, W2_2, b2_2, W2_3, b2_3,
           Wl0, bl0, Wl1, bl1, Wl2, bl2, Wc, bc):
    p0 = _order(X0, L0_indices, L0_values, batch0,
                W0_1, b0_1, W0_2, b0_2, W0_3, b0_3, Wl0, _CFG[0])
    p1 = _order(X1, L1_indices, L1_values, batch1,
                W1_1, b1_1, W1_2, b1_2, W1_3, b1_3, Wl1, _CFG[1])
    p2 = _order(X2, L2_indices, L2_values, batch2,
                W2_1, b2_1, W2_2, b2_2, W2_3, b2_3, Wl2, _CFG[2])
    return _final(p0, p1, p2, bl0, bl1, bl2, Wc, bc)


# speedup vs baseline: 11.7090x; 1.1927x over previous
import functools

import jax
import jax.numpy as jnp
from jax import lax
from jax.experimental import pallas as pl
from jax.experimental.pallas import tpu as pltpu
from jax.experimental.pallas import tpu_sc as plsc

N0, N1, N2 = 100000, 400000, 200000
G = 128
E = 640
EG = E // 16
ZB = 128


def _chunk_starts(total, size):
    starts = []
    i = 0
    while i + size <= total:
        starts.append(i)
        i += size
    if i < total:
        starts.append(total - size)
    return starts


@functools.lru_cache(maxsize=None)
def _make_spmm(n, nnz, d, r, p):
    assert n == 2 * p * r and nnz % E == 0
    b_tot = nnz // E
    nb = -(-b_tot // 16)
    stripe = r // 16
    nh = d // 16
    mesh = plsc.VectorSubcoreMesh(core_axis_name="c", subcore_axis_name="s")

    nk = E // 128

    @functools.partial(
        pl.kernel,
        out_type=jax.ShapeDtypeStruct((n, d), jnp.float32),
        mesh=mesh,
        scratch_types=[
            pltpu.VMEM((ZB, d), jnp.float32),
            pltpu.VMEM((3, E), jnp.int32),
            pltpu.VMEM((3, E), jnp.int32),
            pltpu.VMEM((E, d), jnp.float32),
            pltpu.VMEM((E, d), jnp.float32),
        ] + [pltpu.VMEM((128,), jnp.int32) for _ in range(2 * nk)] + [
            pltpu.VMEM_SHARED((r, d), jnp.float32),
            pltpu.SemaphoreType.DMA, pltpu.SemaphoreType.DMA,
            pltpu.SemaphoreType.DMA, pltpu.SemaphoreType.DMA,
            pltpu.SemaphoreType.DMA, pltpu.SemaphoreType.DMA,
        ],
        compiler_params=pltpu.CompilerParams(use_tc_tiling_on_sc=False,
                                             needs_layout_passes=False),
    )
    def spmm(x_hbm, e3_hbm, out_hbm, zbuf, *rest):
        ebuf = rest[0:2]
        rowsb = rest[2:4]
        lidxs = (rest[4:4 + nk], rest[4 + nk:4 + 2 * nk])
        acc = rest[4 + 2 * nk]
        esem = rest[5 + 2 * nk:7 + 2 * nk]
        gsem = rest[7 + 2 * nk:9 + 2 * nk]
        ssem = rest[9 + 2 * nk:11 + 2 * nk]
        cid = lax.axis_index("c")
        sid = lax.axis_index("s")

        def zero_zbuf(i, _):
            for h in range(nh):
                zbuf[i, pl.ds(16 * h, 16)] = jnp.zeros((16,), jnp.float32)
            return _
        lax.fori_loop(0, ZB, zero_zbuf, None)

        nchunk = r // ZB
        rounds = -(-nchunk // 16)
        tail = r % ZB != 0

        def sweep(lo, flush):
            def rnd(j, _):
                ch = j * 16 + sid

                @pl.when(ch < nchunk)
                def _():
                    st = pl.multiple_of(ch * ZB, ZB)
                    if flush:
                        pltpu.sync_copy(
                            acc.at[pl.ds(st, ZB)],
                            out_hbm.at[pl.ds(pl.multiple_of(lo + st, 8), ZB)])
                    else:
                        pltpu.sync_copy(zbuf, acc.at[pl.ds(st, ZB)])
                return _
            lax.fori_loop(0, rounds, rnd, None)
            if tail:
                @pl.when(sid == 0)
                def _():
                    st = r - ZB
                    if flush:
                        pltpu.sync_copy(
                            acc.at[pl.ds(st, ZB)],
                            out_hbm.at[pl.ds(pl.multiple_of(lo + st, 8), ZB)])
                    else:
                        pltpu.sync_copy(zbuf, acc.at[pl.ds(st, ZB)])

        def valid(j):
            return (j >= 0) & (j * 16 + sid < b_tot)

        def start_edges(j, q):
            @pl.when(valid(j))
            def _():
                pltpu.async_copy(e3_hbm.at[j * 16 + sid], ebuf[q], esem[q])

        def wait_edges(j, q):
            @pl.when(valid(j))
            def _():
                pltpu.make_async_copy(e3_hbm.at[0], ebuf[q], esem[q]).wait()

        def start_gather(j, q):
            @pl.when(valid(j))
            def _():
                pltpu.async_copy(x_hbm.at[ebuf[q].at[1]], rowsb[q], gsem[q])

        def wait_gather(j, q):
            @pl.when(valid(j))
            def _():
                pltpu.make_async_copy(x_hbm.at[ebuf[q].at[1]], rowsb[q],
                                      gsem[q]).wait()

        def wait_scatter(j, q, acc_at):
            @pl.when(valid(j))
            def _():
                for k in range(nk):
                    pltpu.make_async_copy(rowsb[q].at[pl.ds(k * 128, 128)],
                                          acc_at(q, k), ssem[q]).wait()

        def make_compute(lo):
            def compute(j, q):
                @pl.when(valid(j))
                def _():
                    for k in range(nk):
                        def group(g2, _, k=k):
                            g = k * 8 + g2
                            r16 = ebuf[q][0, pl.ds(g * 16, 16)]
                            v16 = plsc.bitcast(ebuf[q][2, pl.ds(g * 16, 16)],
                                               jnp.float32)
                            inw = (r16 >= lo) & (r16 < lo + r)
                            v16 = jnp.where(inw, v16, 0.0)
                            loc = jnp.where(inw, r16 - lo, r16 & 8191)
                            lidxs[q][k][pl.ds(g2 * 16, 16)] = loc
                            for e in range(16):
                                vb = jnp.broadcast_to(v16[e], (16,))
                                ge = g * 16 + e
                                for h in range(nh):
                                    sl = pl.ds(16 * h, 16)
                                    rowsb[q][ge, sl] = rowsb[q][ge, sl] * vb
                            return _
                        lax.fori_loop(0, 8, group, None)
                        pltpu.async_copy(rowsb[q].at[pl.ds(k * 128, 128)],
                                         acc.at[lidxs[q][k]], ssem[q],
                                         add=True)
            return compute

        np_ = nb // 2 + 1

        for pp in range(p):
            win = 2 * pp + cid
            lo = win * r
            sweep(lo, flush=False)
            plsc.subcore_barrier()
            compute = make_compute(lo)

            def acc_at(q, k):
                return acc.at[lidxs[q][k]]

            def half(j, pq, oq):
                wait_edges(j, pq)
                wait_scatter(j - 2, pq, acc_at)
                start_gather(j, pq)
                wait_gather(j - 1, oq)
                compute(j - 1, oq)
                start_edges(j + 1, oq)

            start_edges(0, 0)

            def piter(i, _):
                half(2 * i, 0, 1)
                half(2 * i + 1, 1, 0)
                return _
            lax.fori_loop(0, np_, piter, None)
            jmax = 2 * np_ - 1
            wait_scatter(jmax - 1, 0, acc_at)
            wait_scatter(jmax, 1, acc_at)
            plsc.subcore_barrier()
            sweep(lo, flush=True)
            plsc.subcore_barrier()

    return spmm


_BT = 2000


def _dense_lr(ts, w, b):
    n = ts[0].shape[0]
    din = 16 * len(ts)
    dout = w.shape[1]

    def body(*refs):
        t_refs, w_ref, b_ref = refs[:len(ts)], refs[len(ts)], refs[len(ts) + 1]
        olo, ohi = refs[len(ts) + 2], refs[len(ts) + 3]
        y = b_ref[...]
        for k, t_ref in enumerate(t_refs):
            y = y + jnp.dot(t_ref[...], w_ref[pl.ds(16 * k, 16), :],
                            preferred_element_type=jnp.float32)
        y = jnp.where(y >= 0, y, 0.01 * y)
        olo[...] = y[:, 0:16]
        ohi[...] = y[:, 16:32]

    return pl.pallas_call(
        body,
        grid=(n // _BT,),
        in_specs=[pl.BlockSpec((_BT, 16), lambda i: (i, 0)) for _ in ts] + [
            pl.BlockSpec((din, dout), lambda i: (0, 0)),
            pl.BlockSpec((1, dout), lambda i: (0, 0)),
        ],
        out_specs=[pl.BlockSpec((_BT, 16), lambda i: (i, 0))] * 2,
        out_shape=[jax.ShapeDtypeStruct((n, 16), jnp.float32)] * 2,
    )(*ts, w, b[None, :])


def _pool_proj(hs, wl, batch2d):
    n = hs[0].shape[0]

    def body(*refs):
        h_refs, b_ref, wl_ref, o_ref = refs[:6], refs[6], refs[7], refs[8]
        i = pl.program_id(0)
        y = jnp.zeros((_BT, 10), jnp.float32)
        for k, h_ref in enumerate(h_refs):
            y = y + jnp.dot(h_ref[...], wl_ref[pl.ds(16 * k, 16), :],
                            preferred_element_type=jnp.float32)
        yext = jnp.concatenate(
            [y, jnp.ones((_BT, 1), jnp.float32),
             jnp.zeros((_BT, 5), jnp.float32)], axis=1)
        onehot = (b_ref[...] == lax.broadcasted_iota(jnp.int32, (1, G), 1))
        contrib = lax.dot_general(onehot.astype(jnp.float32), yext,
                                  (((0,), (0,)), ((), ())),
                                  preferred_element_type=jnp.float32)

        @pl.when(i == 0)
        def _():
            o_ref[...] = jnp.zeros_like(o_ref)
        o_ref[...] += contrib

    return pl.pallas_call(
        body,
        grid=(n // _BT,),
        in_specs=[pl.BlockSpec((_BT, 16), lambda i: (i, 0)) for _ in hs] + [
            pl.BlockSpec((_BT, 1), lambda i: (i, 0)),
            pl.BlockSpec((96, 10), lambda i: (0, 0)),
        ],
        out_specs=pl.BlockSpec((G, 16), lambda i: (0, 0)),
        out_shape=jax.ShapeDtypeStruct((G, 16), jnp.float32),
    )(*hs, batch2d, wl)


def _final(p0, p1, p2, bl0, bl1, bl2, wc, bc):
    def body(p0_ref, p1_ref, p2_ref, bl_ref, wc_ref, bc_ref, o_ref):
        ms = []
        for k, p_ref in enumerate((p0_ref, p1_ref, p2_ref)):
            s = p_ref[:, 0:10]
            cnt = p_ref[:, 10:11]
            ms.append((s + cnt * bl_ref[k:k + 1, :]) / jnp.maximum(cnt, 1.0))
        feats = jnp.concatenate(ms, axis=1)
        logits = jnp.dot(feats, wc_ref[...],
                         preferred_element_type=jnp.float32) + bc_ref[...]
        z = logits - jnp.max(logits, axis=1, keepdims=True)
        ez = jnp.exp(z)
        o_ref[...] = ez / jnp.sum(ez, axis=1, keepdims=True)

    bl = jnp.stack([bl0, bl1, bl2], axis=0)
    return pl.pallas_call(
        body,
        out_shape=jax.ShapeDtypeStruct((G, 10), jnp.float32),
    )(p0, p1, p2, bl, wc, bc[None, :])


_CFG = {
    0: (N0, 1600000, 50000, 1),
    1: (N1, 2000000, 100000, 2),
    2: (N2, 800000, 100000, 1),
}


def _order(x, l_idx, l_val, batch, w1, b1, w2, b2, w3, b3, wl, cfg):
    n, nnz, r16, p16 = cfg
    nbt = nnz // E
    rows = l_idx[0].astype(jnp.int32).reshape(nbt, E)
    cols = l_idx[1].astype(jnp.int32).reshape(nbt, E)
    vals = lax.bitcast_convert_type(l_val, jnp.int32).reshape(nbt, E)
    e3 = jnp.stack([rows, cols, vals], axis=1)
    spmm16 = _make_spmm(n, nnz, 16, r16, p16)
    t1 = spmm16(x, e3)
    h1l, h1h = _dense_lr([t1], w1, b1)
    t2l, t2h = spmm16(h1l, e3), spmm16(h1h, e3)
    h2l, h2h = _dense_lr([t2l, t2h], w2, b2)
    t3l, t3h = spmm16(h2l, e3), spmm16(h2h, e3)
    h3l, h3h = _dense_lr([t3l, t3h], w3, b3)
    return _pool_proj([h1l, h1h, h2l, h2h, h3l, h3h], wl,
                      batch[:, None].astype(jnp.int32))


def kernel(X0, X1, X2, L0_indices, L0_values, L1_indices, L1_values,
           L2_indices, L2_values, batch0, batch1, batch2,
           W0_1, b0_1, W0_2, b0_2, W0_3, b0_3,
           W1_1, b1_1, W1_2, b1_2, W1_3, b1_3,
           W2_1, b2_1, W2_2, b2_2, W2_3, b2_3,
           Wl0, bl0, Wl1, bl1, Wl2, bl2, Wc, bc):
    p0 = _order(X0, L0_indices, L0_values, batch0,
                W0_1, b0_1, W0_2, b0_2, W0_3, b0_3, Wl0, _CFG[0])
    p1 = _order(X1, L1_indices, L1_values, batch1,
                W1_1, b1_1, W1_2, b1_2, W1_3, b1_3, Wl1, _CFG[1])
    p2 = _order(X2, L2_indices, L2_values, batch2,
                W2_1, b2_1, W2_2, b2_2, W2_3, b2_3, Wl2, _CFG[2])
    return _final(p0, p1, p2, bl0, bl1, bl2, Wc, bc)
